# baseline (speedup 1.0000x reference)
import functools

import jax
import jax.numpy as jnp
from jax import lax
from jax.experimental import pallas as pl
from jax.experimental.pallas import tpu as pltpu
from jax.experimental.pallas import tpu_sc as plsc

B = 16384
R = 32
NC = 2
NS = 16
L = 16
NW = NC * NS
BPW = B // NW
CH = 128
NCH = BPW // CH
UNROLL = 4


def _lane_take(x, idx):
    dnums = lax.GatherDimensionNumbers(
        offset_dims=(), collapsed_slice_dims=(0,), start_index_map=(0,))
    return lax.gather(x, idx[:, None], dnums, (1,),
                      mode=lax.GatherScatterMode.PROMISE_IN_BOUNDS)


def _body(u_hbm, i_hbm, j_hbm, U_hbm, V_hbm, out_hbm,
          idx_u, idx_i, idx_j, rows_u, rows_i, rows_j, out_v, sem):
    wid = lax.axis_index("s") * NC + lax.axis_index("c")
    base = wid * BPW

    for c in range(NCH):
        off = base + c * CH
        pltpu.sync_copy(u_hbm.at[pl.ds(off, CH)], idx_u.at[c])
        pltpu.sync_copy(i_hbm.at[pl.ds(off, CH)], idx_i.at[c])
        pltpu.sync_copy(j_hbm.at[pl.ds(off, CH)], idx_j.at[c])

    copies = []
    for c in range(NCH):
        dst = pl.ds(c * CH, CH)
        copies.append(pltpu.async_copy(U_hbm.at[idx_u.at[c]], rows_u.at[dst], sem))
        copies.append(pltpu.async_copy(V_hbm.at[idx_i.at[c]], rows_i.at[dst], sem))
        copies.append(pltpu.async_copy(V_hbm.at[idx_j.at[c]], rows_j.at[dst], sem))
    for cp in copies:
        cp.wait()

    lane = lax.iota(jnp.int32, L)
    rots = [(lane + off) & (L - 1) for off in (8, 4, 2, 1)]
    zero = jnp.zeros((L,), jnp.float32)

    def row_block(g, carry):
        acc = zero
        for t in range(L):
            r = g * L + t
            u0 = rows_u[r, pl.ds(0, L)]
            u1 = rows_u[r, pl.ds(L, L)]
            d0 = rows_i[r, pl.ds(0, L)] - rows_j[r, pl.ds(0, L)]
            d1 = rows_i[r, pl.ds(L, L)] - rows_j[r, pl.ds(L, L)]
            s = u0 * d0 + u1 * d1
            for rot in rots:
                s = s + _lane_take(s, rot)
            acc = jnp.where(lane == t, s, acc)
        out_v[pl.ds(g * L, L)] = acc
        return carry

    lax.fori_loop(0, BPW // L, row_block, 0)

    def sig_block(v, carry):
        s = pl.ds(v * L, L)
        x = out_v[s]
        out_v[s] = 1.0 / (1.0 + jnp.exp(-x))
        return carry

    lax.fori_loop(0, BPW // L, sig_block, 0)

    pltpu.sync_copy(out_v, out_hbm.at[pl.ds(base, BPW)])


@jax.jit
def _run(u, i, j, U, V):
    mesh = plsc.VectorSubcoreMesh(core_axis_name="c", subcore_axis_name="s")
    f = functools.partial(
        pl.kernel,
        mesh=mesh,
        out_type=jax.ShapeDtypeStruct((B,), jnp.float32),
        scratch_types=[
            pltpu.VMEM((NCH, CH), jnp.int32),
            pltpu.VMEM((NCH, CH), jnp.int32),
            pltpu.VMEM((NCH, CH), jnp.int32),
            pltpu.VMEM((BPW, R), jnp.float32),
            pltpu.VMEM((BPW, R), jnp.float32),
            pltpu.VMEM((BPW, R), jnp.float32),
            pltpu.VMEM((BPW,), jnp.float32),
            pltpu.SemaphoreType.DMA,
        ],
        compiler_params=pltpu.CompilerParams(use_tc_tiling_on_sc=False),
    )(_body)
    return f(u, i, j, U, V)


def kernel(u, i, j, U, V):
    return _run(u.astype(jnp.int32), i.astype(jnp.int32), j.astype(jnp.int32),
                U, V)
